# baseline (speedup 1.0000x reference)
import functools

import jax
import jax.numpy as jnp
from jax import lax
from jax.experimental import pallas as pl
from jax.experimental.pallas import tpu as pltpu
from jax.experimental.pallas import tpu_sc as plsc

DIM = 64
NC = 2
NS = 16
NW = NC * NS

B_TOTAL = 16384 * 20
CHUNK = 128
GROUP = 4
B_PER_W = B_TOTAL // NW
CHUNKS_PER_W = B_PER_W // CHUNK
GROUPS_PER_W = CHUNKS_PER_W // GROUP
CHUNKS_TOTAL = B_TOTAL // CHUNK

_mesh = plsc.VectorSubcoreMesh(core_axis_name="c", subcore_axis_name="s")


@functools.partial(
    pl.kernel,
    out_type=jax.ShapeDtypeStruct((CHUNKS_TOTAL, CHUNK, DIM), jnp.float32),
    mesh=_mesh,
    scratch_types=[
        pltpu.VMEM((CHUNKS_PER_W, CHUNK), jnp.int32),
        pltpu.VMEM((2, GROUP, CHUNK, DIM), jnp.float32),
        pltpu.SemaphoreType.DMA,
        pltpu.SemaphoreType.DMA,
        pltpu.SemaphoreType.DMA,
    ],
    compiler_params=pltpu.CompilerParams(use_tc_tiling_on_sc=False),
)
def _embed(x_hbm, table_hbm, out_hbm, idx_v, rows_v, gsem, wsem0, wsem1):
    wid = lax.axis_index("s") * NC + lax.axis_index("c")
    pltpu.sync_copy(x_hbm.at[pl.ds(wid * CHUNKS_PER_W, CHUNKS_PER_W)], idx_v)

    wsems = (wsem0, wsem1)

    def run_group(g, db):
        wsem = wsems[db]

        @pl.when(g >= 2)
        def _():
            pltpu.make_async_copy(
                rows_v.at[db], out_hbm.at[pl.ds(0, GROUP)], wsem
            ).wait()

        copies = [
            pltpu.async_copy(
                table_hbm.at[idx_v.at[g * GROUP + b]], rows_v.at[db, b], gsem
            )
            for b in range(GROUP)
        ]
        for c in copies:
            c.wait()

        pltpu.async_copy(
            rows_v.at[db],
            out_hbm.at[pl.ds((wid * GROUPS_PER_W + g) * GROUP, GROUP)],
            wsem,
        )

    def pair(i, carry):
        for db in range(2):
            run_group(i * 2 + db, db)
        return carry

    lax.fori_loop(0, GROUPS_PER_W // 2, pair, 0)

    for db in range(2):
        pltpu.make_async_copy(
            rows_v.at[db], out_hbm.at[pl.ds(0, GROUP)], wsems[db]
        ).wait()


def kernel(x, table):
    xf = x.astype(jnp.int32).reshape(CHUNKS_TOTAL, CHUNK)
    out = _embed(xf, table)
    return out.reshape(x.shape[0], x.shape[1], DIM)
